# baseline (speedup 1.0000x reference)
import jax
import jax.numpy as jnp
from jax.experimental import pallas as pl

D = 768
P, Q, S = 7, 1, 69
N = 5
ROW = (P + Q + S) * D


def _concat_body(p_ref, q_ref, s_ref, o_ref):
    o_ref[:, : P * D] = p_ref[...]
    o_ref[:, P * D : (P + Q) * D] = q_ref[...]
    o_ref[:, (P + Q) * D :] = s_ref[...]


def kernel(embedding_prefix, learnable_quality, embedding_suffix):
    p = embedding_prefix.reshape(N, P * D)
    q = learnable_quality.reshape(N, Q * D)
    s = embedding_suffix.reshape(N, S * D)
    out = pl.pallas_call(
        _concat_body,
        out_shape=jax.ShapeDtypeStruct((N, ROW), jnp.float32),
    )(p, q, s)
    return out.reshape(N, P + Q + S, D)
